# Initial kernel scaffold; baseline (speedup 1.0000x reference)
import jax
import jax.numpy as jnp
from jax.experimental import pallas as pl


def kernel(x, W, b):
    raise NotImplementedError("write your pallas kernel here")


# speedup vs baseline: 2.8224x; 2.8224x over previous
import functools
import jax
import jax.numpy as jnp
from jax.experimental import pallas as pl
from jax.experimental.pallas import tpu as pltpu

IMG = 512
GRID = 64
NA = 3
NC = 80
K_CAND = 1024
MAX_DET = 300
CONF_THRES = 0.001
IOU_THRES = 0.45
MAX_WH = 4096.0
NBOX = GRID * GRID * NA


def _head_kernel(x_ref, w_ref, b_ref, r_ref, p_ref,
                 conf_ref, cls_ref, x1_ref, y1_ref, x2_ref, y2_ref):
    hi = jax.lax.Precision.HIGHEST
    u = []
    for c in range(3):
        xc = x_ref[0, c]
        t = jnp.dot(r_ref[...], xc, precision=hi)
        uc = jnp.dot(t, p_ref[...], precision=hi) * (1.0 / 64.0)
        u.append(uc)

    def feat(f):
        acc = u[0] * w_ref[0:1, f:f + 1]
        acc = acc + u[1] * w_ref[1:2, f:f + 1]
        acc = acc + u[2] * w_ref[2:3, f:f + 1]
        return acc + b_ref[0:1, f:f + 1]

    sig = jax.nn.sigmoid
    for a in range(NA):
        o = a * (5 + NC)
        cx = sig(feat(o + 0)) * float(IMG)
        cy = sig(feat(o + 1)) * float(IMG)
        w = sig(feat(o + 2)) * (float(IMG) / 4.0)
        h = sig(feat(o + 3)) * (float(IMG) / 4.0)
        obj = sig(feat(o + 4))
        best = sig(feat(o + 5)) * obj
        bidx = jnp.zeros((GRID, GRID), jnp.float32)
        for k in range(1, NC):
            s = sig(feat(o + 5 + k)) * obj
            gt = s > best
            best = jnp.where(gt, s, best)
            bidx = jnp.where(gt, float(k), bidx)
        valid = (obj > CONF_THRES) & (best > CONF_THRES)
        conf_ref[0, a] = jnp.where(valid, best, -1.0)
        cls_ref[0, a] = bidx
        x1_ref[0, a] = cx - w * 0.5
        y1_ref[0, a] = cy - h * 0.5
        x2_ref[0, a] = cx + w * 0.5
        y2_ref[0, a] = cy + h * 0.5


def _nms_kernel(scores_ref, bT_ref, boxes_ref, clsr_ref, clsc_ref,
                ob_ref, os_ref, ol_ref, iou_scr, keep_scr):
    scores = scores_ref[0, 0:1, :]
    offr = clsr_ref[0, 0:1, :] * MAX_WH
    x1r = bT_ref[0, 0:1, :] + offr
    y1r = bT_ref[0, 1:2, :] + offr
    x2r = bT_ref[0, 2:3, :] + offr
    y2r = bT_ref[0, 3:4, :] + offr
    area_r = (x2r - x1r) * (y2r - y1r)

    CH = 128
    for ci in range(K_CAND // CH):
        sl = pl.ds(ci * CH, CH)
        offc = clsc_ref[0, sl, :] * MAX_WH
        x1c = boxes_ref[0, sl, 0:1] + offc
        y1c = boxes_ref[0, sl, 1:2] + offc
        x2c = boxes_ref[0, sl, 2:3] + offc
        y2c = boxes_ref[0, sl, 3:4] + offc
        area_c = (x2c - x1c) * (y2c - y1c)
        ltx = jnp.maximum(x1c, x1r)
        lty = jnp.maximum(y1c, y1r)
        rbx = jnp.minimum(x2c, x2r)
        rby = jnp.minimum(y2c, y2r)
        wi = jnp.clip(rbx - ltx, 0.0)
        hi_ = jnp.clip(rby - lty, 0.0)
        inter = wi * hi_
        iou_scr[sl, :] = inter / (area_c + area_r - inter + 1e-7)

    keep_scr[0:1, :] = jnp.where(scores > CONF_THRES, 1.0, 0.0)

    ob_ref[0] = jnp.zeros((MAX_DET, 4), jnp.float32)
    os_ref[0] = jnp.zeros((MAX_DET, 1), jnp.float32)
    ol_ref[0] = jnp.zeros((MAX_DET, 1), jnp.float32)

    iota = jax.lax.broadcasted_iota(jnp.int32, (1, K_CAND), 1)

    def body(i, r):
        onehot = iota == i
        keep = keep_scr[0:1, :]
        keep_i = jnp.sum(jnp.where(onehot, keep, 0.0))
        row = iou_scr[pl.ds(i, 1), :]
        sup = (row > IOU_THRES) & (iota > i)
        keep_scr[0:1, :] = keep * (1.0 - keep_i * sup.astype(jnp.float32))
        kept = (keep_i > 0.5) & (r < MAX_DET)

        @pl.when(kept)
        def _():
            sc = jnp.sum(jnp.where(onehot, scores_ref[0, 0:1, :], 0.0))
            cl = jnp.sum(jnp.where(onehot, clsr_ref[0, 0:1, :], 0.0))
            ob_ref[0, pl.ds(r, 1), :] = boxes_ref[0, pl.ds(i, 1), :]
            os_ref[0, pl.ds(r, 1), :] = sc.reshape(1, 1)
            ol_ref[0, pl.ds(r, 1), :] = cl.reshape(1, 1)

        return r + kept.astype(jnp.int32)

    jax.lax.fori_loop(0, K_CAND, body, jnp.int32(0))

    b = jnp.clip(ob_ref[0], 0.0, float(IMG))
    s = os_ref[0]
    cl = ol_ref[0]
    x1, y1 = b[:, 0:1], b[:, 1:2]
    x2, y2 = b[:, 2:3], b[:, 3:4]
    size_ok = (x2 > x1) & (y2 > y1) & ((x2 - x1) >= 1.0) & ((y2 - y1) >= 1.0)
    valid = (s > 0.0) & size_ok
    vf = valid.astype(jnp.float32)
    xmin = jnp.minimum(x1, x2)
    xmax = jnp.maximum(x1, x2)
    ymin = jnp.minimum(y1, y2)
    ymax = jnp.maximum(y1, y2)
    ob_ref[0] = jnp.concatenate([xmin, ymin, xmax, ymax], axis=1) * vf
    os_ref[0] = s * vf
    ol_ref[0] = jnp.clip(cl, 0.0, 19.0) * vf


@jax.jit
def kernel(x, W, b):
    B = x.shape[0]
    eye = jnp.eye(GRID, dtype=jnp.float32)
    R = jnp.repeat(eye, 8, axis=1)
    P = R.T
    b2 = b.reshape(1, NA * (5 + NC))

    if True:
        pooled = x.reshape(B, 3, GRID, 8, GRID, 8).mean(axis=(3, 5))
        feat = jnp.einsum('bchw,cf->bhwf', pooled, W) + b
        p = feat.reshape(B, GRID * GRID * NA, 5 + NC)
        xy = jax.nn.sigmoid(p[..., 0:2]) * float(IMG)
        wh = jax.nn.sigmoid(p[..., 2:4]) * (float(IMG) / 4.0)
        obj = jax.nn.sigmoid(p[..., 4])
        cls = jax.nn.sigmoid(p[..., 5:])
        sc_all = cls * obj[..., None]
        best = sc_all.max(axis=-1)
        bidx = jnp.argmax(sc_all, axis=-1).astype(jnp.float32)
        validm = (obj > CONF_THRES) & (best > CONF_THRES)
        conf_all = jnp.where(validm, best, -1.0)
        x1a = xy[..., 0] - wh[..., 0] * 0.5
        y1a = xy[..., 1] - wh[..., 1] * 0.5
        x2a = xy[..., 0] + wh[..., 0] * 0.5
        y2a = xy[..., 1] + wh[..., 1] * 0.5
        top_scores, top_idx = jax.lax.top_k(conf_all, K_CAND)
        gsel = lambda a: jnp.take_along_axis(a, top_idx, axis=1)
        x1s, y1s, x2s, y2s = gsel(x1a), gsel(y1a), gsel(x2a), gsel(y2a)
        cls_sel = gsel(bidx)
        boxes_sel = jnp.stack([x1s, y1s, x2s, y2s], axis=2)
    del R, P, b2
    boxesT_sel = jnp.stack([x1s, y1s, x2s, y2s], axis=1)
    cls_col = cls_sel.reshape(B, K_CAND, 1)

    if False:
        def one(ts, bx, cl):
            cand_valid = ts > CONF_THRES
            off = cl * MAX_WH
            ob = bx + off[:, None]
            area = (ob[:, 2] - ob[:, 0]) * (ob[:, 3] - ob[:, 1])
            lt = jnp.maximum(ob[:, None, :2], ob[None, :, :2])
            rb = jnp.minimum(ob[:, None, 2:], ob[None, :, 2:])
            whi = jnp.clip(rb - lt, 0.0)
            inter = whi[..., 0] * whi[..., 1]
            iou = inter / (area[:, None] + area[None, :] - inter + 1e-7)
            idxs = jnp.arange(K_CAND)

            def body(i, keep):
                sup = (iou[i] > IOU_THRES) & keep[i] & (idxs > i)
                return keep & (~sup)

            keep = jax.lax.fori_loop(0, K_CAND, body, cand_valid)
            final_m = jnp.where(keep, ts, -1.0)
            det_scores, det_idx = jax.lax.top_k(final_m, MAX_DET)
            det_boxes = bx[det_idx]
            det_cls = cl[det_idx]
            det_valid = det_scores > 0.0
            bcl = jnp.clip(det_boxes, 0.0, float(IMG))
            x1, y1, x2, y2 = bcl[:, 0], bcl[:, 1], bcl[:, 2], bcl[:, 3]
            size_ok = (x2 > x1) & (y2 > y1) & ((x2 - x1) >= 1.0) & ((y2 - y1) >= 1.0)
            det_valid = det_valid & size_ok
            vf = det_valid.astype(jnp.float32)
            obx = jnp.stack([jnp.minimum(x1, x2), jnp.minimum(y1, y2),
                             jnp.maximum(x1, x2), jnp.maximum(y1, y2)], axis=1)
            return (obx * vf[:, None], jnp.where(det_valid, det_scores, 0.0),
                    jnp.clip(det_cls, 0.0, 19.0) * vf)

        bb, ss, ll = jax.vmap(one)(top_scores, boxes_sel, cls_sel)
        return bb, ll.astype(jnp.int32), ss

    out_b, out_s, out_l = pl.pallas_call(
        _nms_kernel,
        grid=(B,),
        in_specs=[
            pl.BlockSpec((1, 1, K_CAND), lambda i: (i, 0, 0)),
            pl.BlockSpec((1, 4, K_CAND), lambda i: (i, 0, 0)),
            pl.BlockSpec((1, K_CAND, 4), lambda i: (i, 0, 0)),
            pl.BlockSpec((1, 1, K_CAND), lambda i: (i, 0, 0)),
            pl.BlockSpec((1, K_CAND, 1), lambda i: (i, 0, 0)),
        ],
        out_specs=[
            pl.BlockSpec((1, MAX_DET, 4), lambda i: (i, 0, 0)),
            pl.BlockSpec((1, MAX_DET, 1), lambda i: (i, 0, 0)),
            pl.BlockSpec((1, MAX_DET, 1), lambda i: (i, 0, 0)),
        ],
        out_shape=[
            jax.ShapeDtypeStruct((B, MAX_DET, 4), jnp.float32),
            jax.ShapeDtypeStruct((B, MAX_DET, 1), jnp.float32),
            jax.ShapeDtypeStruct((B, MAX_DET, 1), jnp.float32),
        ],
        scratch_shapes=[
            pltpu.VMEM((K_CAND, K_CAND), jnp.float32),
            pltpu.VMEM((1, K_CAND), jnp.float32),
        ],
    )(top_scores.reshape(B, 1, K_CAND), boxesT_sel, boxes_sel,
      cls_sel.reshape(B, 1, K_CAND), cls_col)

    boxes = out_b
    scores = out_s.reshape(B, MAX_DET)
    labels = out_l.reshape(B, MAX_DET).astype(jnp.int32)
    return boxes, labels, scores


# speedup vs baseline: 2.8226x; 1.0001x over previous
import jax
import jax.numpy as jnp
from jax.experimental import pallas as pl
from jax.experimental.pallas import tpu as pltpu

IMG = 512
GRID = 64
NA = 3
NC = 80
K_CAND = 1024
MAX_DET = 300
CONF_THRES = 0.001
IOU_THRES = 0.45
MAX_WH = 4096.0


def _nms_kernel(scores_ref, bT_ref, boxes_ref, clsr_ref, clsc_ref,
                ob_ref, os_ref, ol_ref, iou_scr, keep_scr):
    scores = scores_ref[0, 0:1, :]
    offr = clsr_ref[0, 0:1, :] * MAX_WH
    x1r = bT_ref[0, 0:1, :] + offr
    y1r = bT_ref[0, 1:2, :] + offr
    x2r = bT_ref[0, 2:3, :] + offr
    y2r = bT_ref[0, 3:4, :] + offr
    area_r = (x2r - x1r) * (y2r - y1r)

    CH = 128
    for ci in range(K_CAND // CH):
        sl = pl.ds(ci * CH, CH)
        offc = clsc_ref[0, sl, :] * MAX_WH
        x1c = boxes_ref[0, sl, 0:1] + offc
        y1c = boxes_ref[0, sl, 1:2] + offc
        x2c = boxes_ref[0, sl, 2:3] + offc
        y2c = boxes_ref[0, sl, 3:4] + offc
        area_c = (x2c - x1c) * (y2c - y1c)
        ltx = jnp.maximum(x1c, x1r)
        lty = jnp.maximum(y1c, y1r)
        rbx = jnp.minimum(x2c, x2r)
        rby = jnp.minimum(y2c, y2r)
        wi = jnp.clip(rbx - ltx, 0.0)
        hi_ = jnp.clip(rby - lty, 0.0)
        inter = wi * hi_
        iou_scr[sl, :] = inter / (area_c + area_r - inter + 1e-7)

    keep_scr[0:1, :] = jnp.where(scores > CONF_THRES, 1.0, 0.0)

    ob_ref[0] = jnp.zeros((MAX_DET, 4), jnp.float32)
    os_ref[0] = jnp.zeros((MAX_DET, 1), jnp.float32)
    ol_ref[0] = jnp.zeros((MAX_DET, 1), jnp.float32)

    iota = jax.lax.broadcasted_iota(jnp.int32, (1, K_CAND), 1)

    def body(i, r):
        onehot = iota == i
        keep = keep_scr[0:1, :]
        keep_i = jnp.sum(jnp.where(onehot, keep, 0.0))
        row = iou_scr[pl.ds(i, 1), :]
        sup = (row > IOU_THRES) & (iota > i)
        keep_scr[0:1, :] = keep * (1.0 - keep_i * sup.astype(jnp.float32))
        kept = (keep_i > 0.5) & (r < MAX_DET)

        @pl.when(kept)
        def _():
            sc = jnp.sum(jnp.where(onehot, scores_ref[0, 0:1, :], 0.0))
            cl = jnp.sum(jnp.where(onehot, clsr_ref[0, 0:1, :], 0.0))
            ob_ref[0, pl.ds(r, 1), :] = boxes_ref[0, pl.ds(i, 1), :]
            os_ref[0, pl.ds(r, 1), :] = sc.reshape(1, 1)
            ol_ref[0, pl.ds(r, 1), :] = cl.reshape(1, 1)

        return r + kept.astype(jnp.int32)

    jax.lax.fori_loop(0, K_CAND, body, jnp.int32(0))

    b = jnp.clip(ob_ref[0], 0.0, float(IMG))
    s = os_ref[0]
    cl = ol_ref[0]
    x1, y1 = b[:, 0:1], b[:, 1:2]
    x2, y2 = b[:, 2:3], b[:, 3:4]
    size_ok = (x2 > x1) & (y2 > y1) & ((x2 - x1) >= 1.0) & ((y2 - y1) >= 1.0)
    valid = (s > 0.0) & size_ok
    vf = valid.astype(jnp.float32)
    xmin = jnp.minimum(x1, x2)
    xmax = jnp.maximum(x1, x2)
    ymin = jnp.minimum(y1, y2)
    ymax = jnp.maximum(y1, y2)
    ob_ref[0] = jnp.concatenate([xmin, ymin, xmax, ymax], axis=1) * vf
    os_ref[0] = s * vf
    ol_ref[0] = jnp.clip(cl, 0.0, 19.0) * vf


@jax.jit
def kernel(x, W, b):
    B = x.shape[0]

    pooled = x.reshape(B, 3, GRID, 8, GRID, 8).mean(axis=(3, 5))
    feat = jnp.einsum('bchw,cf->bhwf', pooled, W) + b
    p = feat.reshape(B, GRID * GRID * NA, 5 + NC)
    xy = jax.nn.sigmoid(p[..., 0:2]) * float(IMG)
    wh = jax.nn.sigmoid(p[..., 2:4]) * (float(IMG) / 4.0)
    obj = jax.nn.sigmoid(p[..., 4])
    cls = jax.nn.sigmoid(p[..., 5:])
    sc_all = cls * obj[..., None]
    best = sc_all.max(axis=-1)
    bidx = jnp.argmax(sc_all, axis=-1).astype(jnp.float32)
    validm = (obj > CONF_THRES) & (best > CONF_THRES)
    conf_all = jnp.where(validm, best, -1.0)
    x1a = xy[..., 0] - wh[..., 0] * 0.5
    y1a = xy[..., 1] - wh[..., 1] * 0.5
    x2a = xy[..., 0] + wh[..., 0] * 0.5
    y2a = xy[..., 1] + wh[..., 1] * 0.5
    top_scores, top_idx = jax.lax.top_k(conf_all, K_CAND)
    gsel = lambda a: jnp.take_along_axis(a, top_idx, axis=1)
    x1s, y1s, x2s, y2s = gsel(x1a), gsel(y1a), gsel(x2a), gsel(y2a)
    cls_sel = gsel(bidx)
    boxes_sel = jnp.stack([x1s, y1s, x2s, y2s], axis=2)
    boxesT_sel = jnp.stack([x1s, y1s, x2s, y2s], axis=1)
    cls_col = cls_sel.reshape(B, K_CAND, 1)

    out_b, out_s, out_l = pl.pallas_call(
        _nms_kernel,
        grid=(B,),
        in_specs=[
            pl.BlockSpec((1, 1, K_CAND), lambda i: (i, 0, 0)),
            pl.BlockSpec((1, 4, K_CAND), lambda i: (i, 0, 0)),
            pl.BlockSpec((1, K_CAND, 4), lambda i: (i, 0, 0)),
            pl.BlockSpec((1, 1, K_CAND), lambda i: (i, 0, 0)),
            pl.BlockSpec((1, K_CAND, 1), lambda i: (i, 0, 0)),
        ],
        out_specs=[
            pl.BlockSpec((1, MAX_DET, 4), lambda i: (i, 0, 0)),
            pl.BlockSpec((1, MAX_DET, 1), lambda i: (i, 0, 0)),
            pl.BlockSpec((1, MAX_DET, 1), lambda i: (i, 0, 0)),
        ],
        out_shape=[
            jax.ShapeDtypeStruct((B, MAX_DET, 4), jnp.float32),
            jax.ShapeDtypeStruct((B, MAX_DET, 1), jnp.float32),
            jax.ShapeDtypeStruct((B, MAX_DET, 1), jnp.float32),
        ],
        scratch_shapes=[
            pltpu.VMEM((K_CAND, K_CAND), jnp.float32),
            pltpu.VMEM((1, K_CAND), jnp.float32),
        ],
    )(top_scores.reshape(B, 1, K_CAND), boxesT_sel, boxes_sel,
      cls_sel.reshape(B, 1, K_CAND), cls_col)

    boxes = out_b
    scores = out_s.reshape(B, MAX_DET)
    labels = out_l.reshape(B, MAX_DET).astype(jnp.int32)
    return boxes, labels, scores
